# Initial kernel scaffold; baseline (speedup 1.0000x reference)
import jax
import jax.numpy as jnp
from jax.experimental import pallas as pl


def kernel(x, W, b):
    raise NotImplementedError("write your pallas kernel here")


# speedup vs baseline: 5.2547x; 5.2547x over previous
import functools

import jax
import jax.numpy as jnp
from jax.experimental import pallas as pl

_K = 16
_HI = jax.lax.Precision.HIGHEST


def _tc_body(x_ref, wnT_ref, wcT_ref, b_ref, out_ref):
    x = x_ref[0]
    N = x.shape[0]
    G = jax.lax.dot_general(x, x, (((1,), (1,)), ((), ())), precision=_HI)
    sq_col = jnp.sum(x * x, axis=1, keepdims=True)
    ii = jax.lax.broadcasted_iota(jnp.int32, (N, N), 0)
    jj = jax.lax.broadcasted_iota(jnp.int32, (N, N), 1)
    eye = ii == jj
    sq_row = jnp.sum(jnp.where(eye, G, 0.0), axis=0, keepdims=True)

    yn = jnp.dot(x, wnT_ref[...], precision=_HI)
    yc = jnp.dot(x, wcT_ref[...], precision=_HI) + b_ref[...]

    neg = 2.0 * G - sq_col - sq_row
    inv_di = 1.0 / jnp.sqrt(sq_col)
    acc = jnp.zeros_like(yc)
    for _ in range(_K):
        m = jnp.max(neg, axis=1, keepdims=True)
        ism = neg == m
        jsel = jnp.min(jnp.where(ism, jj, jnp.int32(1 << 30)), axis=1,
                       keepdims=True)
        sel = jj == jsel
        self_f = sel.astype(jnp.float32)
        g_sel = jnp.sum(G * self_f, axis=1, keepdims=True)
        sq_j = jnp.sum(sq_row * self_f, axis=1, keepdims=True)
        w = g_sel * inv_di / jnp.sqrt(sq_j)
        row = jnp.dot(self_f, yn, precision=_HI)
        acc = jnp.maximum(acc, w * (row + yc))
        neg = jnp.where(sel, -jnp.inf, neg)
    out_ref[0] = acc


def kernel(x, W, b):
    B, N, C = x.shape
    wnT = W[:, :C].T
    wcT = W[:, C:].T
    b2 = b.reshape(1, C)
    out = pl.pallas_call(
        _tc_body,
        grid=(B,),
        in_specs=[
            pl.BlockSpec((1, N, C), lambda i: (i, 0, 0)),
            pl.BlockSpec((C, C), lambda i: (0, 0)),
            pl.BlockSpec((C, C), lambda i: (0, 0)),
            pl.BlockSpec((1, C), lambda i: (0, 0)),
        ],
        out_specs=pl.BlockSpec((1, N, C), lambda i: (i, 0, 0)),
        out_shape=jax.ShapeDtypeStruct((B, N, C), jnp.float32),
    )(x, wnT, wcT, b2)
    return out
